# Initial kernel scaffold; baseline (speedup 1.0000x reference)
import jax
import jax.numpy as jnp
from jax.experimental import pallas as pl


def kernel(features, edge_index, edge_attr, W1, b1, W2, b2, Wd, bd, Wfc, bfc):
    raise NotImplementedError("write your pallas kernel here")


# speedup vs baseline: 8.2058x; 8.2058x over previous
import dataclasses
import functools

import jax
import jax.numpy as jnp
from jax import lax
from jax.experimental import pallas as pl
from jax.experimental.pallas import tpu as pltpu
from jax.experimental.pallas import tpu_sc as plsc

N = 10000
NP = 10240
F = 128
H = 64
E = 320000
NS = 16
NW = 32
EPW = 10240
CH = 128
NCH = EPW // CH
EP = NW * EPW
RPS = NP // NS

_MESH = plsc.VectorSubcoreMesh(
    core_axis_name="c", subcore_axis_name="s", num_cores=2, num_subcores=16
)

_SC_PARAMS = pltpu.CompilerParams()
if "needs_layout_passes" in pltpu.CompilerParams.__dataclass_fields__:
    _SC_PARAMS = dataclasses.replace(_SC_PARAMS, needs_layout_passes=False)
if "use_tc_tiling_on_sc" in pltpu.CompilerParams.__dataclass_fields__:
    _SC_PARAMS = dataclasses.replace(_SC_PARAMS, use_tc_tiling_on_sc=False)


def _zero_vec16():
    return jnp.zeros((16,), jnp.float32)


@functools.partial(
    pl.kernel,
    out_type=jax.ShapeDtypeStruct((NW, NP), jnp.float32),
    mesh=_MESH,
    compiler_params=_SC_PARAMS,
    scratch_types=[
        pltpu.VMEM((NCH, CH), jnp.int32),
        pltpu.VMEM((NCH, CH), jnp.float32),
        pltpu.VMEM((NP,), jnp.float32),
    ],
)
def _sc_deg(dst_hbm, w_hbm, out_hbm, dst_v, w_v, acc_v):
    cid = lax.axis_index("c")
    sid = lax.axis_index("s")
    wid = cid * NS + sid

    @pl.loop(0, NP // 16)
    def _(i):
        acc_v[pl.ds(i * 16, 16)] = _zero_vec16()

    pltpu.sync_copy(dst_hbm.at[wid], dst_v)
    pltpu.sync_copy(w_hbm.at[wid], w_v)

    @pl.loop(0, NCH)
    def _(c):
        for j in range(CH // 16):
            idx = dst_v[c, pl.ds(j * 16, 16)]
            val = w_v[c, pl.ds(j * 16, 16)]
            plsc.addupdate_scatter(acc_v, [idx], val)

    pltpu.sync_copy(acc_v, out_hbm.at[wid])


def _make_conv(dim):
    @functools.partial(
        pl.kernel,
        out_type=jax.ShapeDtypeStruct((2, NP, dim), jnp.float32),
        mesh=_MESH,
        compiler_params=_SC_PARAMS,
        scratch_types=[
            pltpu.VMEM((NCH, CH), jnp.int32),
            pltpu.VMEM((NCH, CH), jnp.int32),
            pltpu.VMEM((NCH, CH), jnp.float32),
            pltpu.VMEM((CH, dim), jnp.float32),
            pltpu.VMEM((CH, dim), jnp.float32),
            pltpu.VMEM_SHARED((NP, dim), jnp.float32),
            pltpu.SemaphoreType.DMA,
            pltpu.SemaphoreType.DMA,
        ],
    )
    def conv(h_hbm, src_hbm, dst_hbm, w_hbm, out_hbm,
             src_v, dst_v, w_v, rows0_v, rows1_v, acc_sh, sem0, sem1):
        cid = lax.axis_index("c")
        sid = lax.axis_index("s")
        wid = cid * NS + sid

        @pl.loop(0, CH)
        def _(r):
            for j in range(dim // 16):
                rows0_v[r, pl.ds(j * 16, 16)] = _zero_vec16()

        for k in range(RPS // CH):
            pltpu.sync_copy(rows0_v, acc_sh.at[pl.ds(sid * RPS + k * CH, CH)])
        plsc.subcore_barrier()

        pltpu.sync_copy(src_hbm.at[wid], src_v)
        pltpu.sync_copy(dst_hbm.at[wid], dst_v)
        pltpu.sync_copy(w_hbm.at[wid], w_v)

        def scale(rows_v, c):
            @pl.loop(0, CH // 16)
            def _(g):
                wv = w_v[c, pl.ds(g * 16, 16)]
                for l in range(16):
                    sv = lax.broadcast(wv[l], (16,))
                    for j in range(dim // 16):
                        sl = pl.ds(j * 16, 16)
                        e = g * 16 + l
                        rows_v[e, sl] = rows_v[e, sl] * sv

        if dim <= 64:
            pltpu.async_copy(h_hbm.at[src_v.at[0]], rows0_v, sem0)
            pltpu.async_copy(h_hbm.at[src_v.at[1]], rows1_v, sem1)

            @pl.loop(0, NCH, step=2)
            def _(c):
                pltpu.make_async_copy(h_hbm.at[src_v.at[c]], rows0_v, sem0).wait()
                scale(rows0_v, c)
                pltpu.sync_copy(rows0_v, acc_sh.at[dst_v.at[c]], add=True)

                @pl.when(c + 2 < NCH)
                def _():
                    pltpu.async_copy(h_hbm.at[src_v.at[c + 2]], rows0_v, sem0)

                pltpu.make_async_copy(h_hbm.at[src_v.at[c + 1]], rows1_v, sem1).wait()
                scale(rows1_v, c + 1)
                pltpu.sync_copy(rows1_v, acc_sh.at[dst_v.at[c + 1]], add=True)

                @pl.when(c + 3 < NCH)
                def _():
                    pltpu.async_copy(h_hbm.at[src_v.at[c + 3]], rows1_v, sem1)
        else:
            @pl.loop(0, NCH)
            def _(c):
                pltpu.async_copy(h_hbm.at[src_v.at[c]], rows0_v, sem0).wait()
                scale(rows0_v, c)
                pltpu.sync_copy(rows0_v, acc_sh.at[dst_v.at[c]], add=True)

        plsc.subcore_barrier()
        for k in range(RPS // CH):
            sl = pl.ds(sid * RPS + k * CH, CH)
            pltpu.sync_copy(acc_sh.at[sl], out_hbm.at[cid].at[sl])

    return conv


_sc_conv64 = _make_conv(64)
_sc_conv128 = _make_conv(128)


@functools.partial(
    pl.kernel,
    out_type=(
        jax.ShapeDtypeStruct((NW, NCH, CH), jnp.float32),
        jax.ShapeDtypeStruct((NW, NP), jnp.float32),
    ),
    mesh=_MESH,
    compiler_params=_SC_PARAMS,
    scratch_types=[
        pltpu.VMEM((NCH, CH), jnp.int32),
        pltpu.VMEM((NCH, CH), jnp.int32),
        pltpu.VMEM((NCH, CH), jnp.float32),
        pltpu.VMEM((CH, F), jnp.float32),
        pltpu.VMEM((CH, F), jnp.float32),
        pltpu.VMEM((CH, F), jnp.float32),
        pltpu.VMEM((CH, F), jnp.float32),
        pltpu.VMEM((NP,), jnp.float32),
        pltpu.SemaphoreType.DMA,
        pltpu.SemaphoreType.DMA,
        pltpu.SemaphoreType.DMA,
        pltpu.SemaphoreType.DMA,
    ],
)
def _sc_decode(z_hbm, src_hbm, dst_hbm, probs_hbm, degp_hbm,
               src_v, dst_v, probs_v, zs0_v, zd0_v, zs1_v, zd1_v,
               acc_v, ss0, sd0, ss1, sd1):
    cid = lax.axis_index("c")
    sid = lax.axis_index("s")
    wid = cid * NS + sid

    @pl.loop(0, NP // 16)
    def _(i):
        acc_v[pl.ds(i * 16, 16)] = _zero_vec16()

    pltpu.sync_copy(src_hbm.at[wid], src_v)
    pltpu.sync_copy(dst_hbm.at[wid], dst_v)

    lane = lax.iota(jnp.int32, 16)
    ebase = wid * EPW

    def dots_chunk(zs_v, zd_v, c):
        @pl.loop(0, CH // 16)
        def _(g):
            dots = _zero_vec16()
            for l in range(16):
                e = g * 16 + l
                prod = zs_v[e, pl.ds(0, 16)] * zd_v[e, pl.ds(0, 16)]
                for j in range(1, F // 16):
                    sl = pl.ds(j * 16, 16)
                    prod = prod + zs_v[e, sl] * zd_v[e, sl]
                dots = dots + jnp.where(lane == l, jnp.sum(prod), 0.0)
            sl = pl.ds(g * 16, 16)
            eid = lane + (ebase + c * CH + g * 16)
            p = jnp.where(eid < E, 1.0 / (1.0 + jnp.exp(-dots)), 0.0)
            probs_v[c, sl] = p
            plsc.addupdate_scatter(acc_v, [dst_v[c, sl]], p)

    pltpu.async_copy(z_hbm.at[src_v.at[0]], zs0_v, ss0)
    pltpu.async_copy(z_hbm.at[dst_v.at[0]], zd0_v, sd0)
    pltpu.async_copy(z_hbm.at[src_v.at[1]], zs1_v, ss1)
    pltpu.async_copy(z_hbm.at[dst_v.at[1]], zd1_v, sd1)

    @pl.loop(0, NCH, step=2)
    def _(c):
        pltpu.make_async_copy(z_hbm.at[src_v.at[c]], zs0_v, ss0).wait()
        pltpu.make_async_copy(z_hbm.at[dst_v.at[c]], zd0_v, sd0).wait()
        dots_chunk(zs0_v, zd0_v, c)

        @pl.when(c + 2 < NCH)
        def _():
            pltpu.async_copy(z_hbm.at[src_v.at[c + 2]], zs0_v, ss0)
            pltpu.async_copy(z_hbm.at[dst_v.at[c + 2]], zd0_v, sd0)

        pltpu.make_async_copy(z_hbm.at[src_v.at[c + 1]], zs1_v, ss1).wait()
        pltpu.make_async_copy(z_hbm.at[dst_v.at[c + 1]], zd1_v, sd1).wait()
        dots_chunk(zs1_v, zd1_v, c + 1)

        @pl.when(c + 3 < NCH)
        def _():
            pltpu.async_copy(z_hbm.at[src_v.at[c + 3]], zs1_v, ss1)
            pltpu.async_copy(z_hbm.at[dst_v.at[c + 3]], zd1_v, sd1)

    pltpu.sync_copy(probs_v, probs_hbm.at[wid])
    pltpu.sync_copy(acc_v, degp_hbm.at[wid])


_BT = 512


def _mm(x, W):
    M, K = x.shape
    _, Nn = W.shape

    def body(x_ref, w_ref, o_ref):
        o_ref[...] = jnp.dot(x_ref[...], w_ref[...],
                             preferred_element_type=jnp.float32)

    return pl.pallas_call(
        body,
        grid=(M // _BT,),
        in_specs=[pl.BlockSpec((_BT, K), lambda i: (i, 0)),
                  pl.BlockSpec((K, Nn), lambda i: (0, 0))],
        out_specs=pl.BlockSpec((_BT, Nn), lambda i: (i, 0)),
        out_shape=jax.ShapeDtypeStruct((M, Nn), jnp.float32),
    )(x, W)


def _norms(degp, h1):
    def body(dp_ref, h1_ref, h1p_ref, dis_ref, invd_ref):
        deg = jnp.sum(dp_ref[...], axis=0, keepdims=True) + 1.0
        dis = lax.rsqrt(deg)
        invd = 1.0 / deg
        dis_ref[...] = dis.T
        invd_ref[...] = invd.T
        h1p_ref[...] = h1_ref[...] * dis.T

    return pl.pallas_call(
        body,
        grid=(NP // _BT,),
        in_specs=[pl.BlockSpec((NW, _BT), lambda i: (0, i)),
                  pl.BlockSpec((_BT, H), lambda i: (i, 0))],
        out_specs=[pl.BlockSpec((_BT, H), lambda i: (i, 0)),
                   pl.BlockSpec((_BT, 1), lambda i: (i, 0)),
                   pl.BlockSpec((_BT, 1), lambda i: (i, 0))],
        out_shape=[jax.ShapeDtypeStruct((NP, H), jnp.float32),
                   jax.ShapeDtypeStruct((NP, 1), jnp.float32),
                   jax.ShapeDtypeStruct((NP, 1), jnp.float32)],
    )(degp, h1)


def _post1(acc, h1, dis, invd, b1, W2):
    def body(acc_ref, h1_ref, dis_ref, invd_ref, b1_ref, w2_ref,
             h2p_ref, h2i_ref):
        s = acc_ref[0] + acc_ref[1]
        h = jnp.maximum(
            dis_ref[...] * s + h1_ref[...] * invd_ref[...] + b1_ref[...], 0.0)
        h2 = jnp.dot(h, w2_ref[...], preferred_element_type=jnp.float32)
        h2p_ref[...] = h2 * dis_ref[...]
        h2i_ref[...] = h2 * invd_ref[...]

    return pl.pallas_call(
        body,
        grid=(NP // _BT,),
        in_specs=[pl.BlockSpec((2, _BT, H), lambda i: (0, i, 0)),
                  pl.BlockSpec((_BT, H), lambda i: (i, 0)),
                  pl.BlockSpec((_BT, 1), lambda i: (i, 0)),
                  pl.BlockSpec((_BT, 1), lambda i: (i, 0)),
                  pl.BlockSpec((1, H), lambda i: (0, 0)),
                  pl.BlockSpec((H, F), lambda i: (0, 0))],
        out_specs=[pl.BlockSpec((_BT, F), lambda i: (i, 0)),
                   pl.BlockSpec((_BT, F), lambda i: (i, 0))],
        out_shape=[jax.ShapeDtypeStruct((NP, F), jnp.float32),
                   jax.ShapeDtypeStruct((NP, F), jnp.float32)],
    )(acc, h1, dis, invd, b1, W2)


def _post2(acc, h2i, dis, b2, x, Wd):
    def body(acc_ref, h2i_ref, dis_ref, b2_ref, x_ref, wd_ref, z_ref, y_ref):
        s = acc_ref[0] + acc_ref[1]
        logits = dis_ref[...] * s + h2i_ref[...] + b2_ref[...]
        z = 1.0 / (1.0 + jnp.exp(-logits))
        z_ref[...] = z
        y_ref[...] = jnp.dot(z + x_ref[...], wd_ref[...],
                             preferred_element_type=jnp.float32)

    return pl.pallas_call(
        body,
        grid=(NP // _BT,),
        in_specs=[pl.BlockSpec((2, _BT, F), lambda i: (0, i, 0)),
                  pl.BlockSpec((_BT, F), lambda i: (i, 0)),
                  pl.BlockSpec((_BT, 1), lambda i: (i, 0)),
                  pl.BlockSpec((1, F), lambda i: (0, 0)),
                  pl.BlockSpec((_BT, F), lambda i: (i, 0)),
                  pl.BlockSpec((F, H), lambda i: (0, 0))],
        out_specs=[pl.BlockSpec((_BT, F), lambda i: (i, 0)),
                   pl.BlockSpec((_BT, H), lambda i: (i, 0))],
        out_shape=[jax.ShapeDtypeStruct((NP, F), jnp.float32),
                   jax.ShapeDtypeStruct((NP, H), jnp.float32)],
    )(acc, h2i, dis, b2, x, Wd)


def _norms2(degp, y):
    def body(dp_ref, y_ref, yp_ref, yi_ref, dis_ref):
        deg = jnp.sum(dp_ref[...], axis=0, keepdims=True) + 1.0
        dis = lax.rsqrt(deg)
        invd = 1.0 / deg
        yp_ref[...] = y_ref[...] * dis.T
        yi_ref[...] = y_ref[...] * invd.T
        dis_ref[...] = dis.T

    return pl.pallas_call(
        body,
        grid=(NP // _BT,),
        in_specs=[pl.BlockSpec((NW, _BT), lambda i: (0, i)),
                  pl.BlockSpec((_BT, H), lambda i: (i, 0))],
        out_specs=[pl.BlockSpec((_BT, H), lambda i: (i, 0)),
                   pl.BlockSpec((_BT, H), lambda i: (i, 0)),
                   pl.BlockSpec((_BT, 1), lambda i: (i, 0))],
        out_shape=[jax.ShapeDtypeStruct((NP, H), jnp.float32),
                   jax.ShapeDtypeStruct((NP, H), jnp.float32),
                   jax.ShapeDtypeStruct((NP, 1), jnp.float32)],
    )(degp, y)


def _head(acc, yi, dis2, bd, Wfc2d, bfc):
    nsteps = NP // _BT

    def body(acc_ref, yi_ref, dis_ref, bd_ref, wfc_ref, bfc_ref, o_ref):
        i = pl.program_id(0)

        @pl.when(i == 0)
        def _():
            o_ref[...] = jnp.zeros((1, 1), jnp.float32)

        s = acc_ref[0] + acc_ref[1]
        hd = jnp.maximum(
            dis_ref[...] * s + yi_ref[...] + bd_ref[...], 0.0)
        o_ref[...] += jnp.sum(hd * wfc_ref[...]).reshape(1, 1)

        @pl.when(i == nsteps - 1)
        def _():
            t = o_ref[...] + bfc_ref[...]
            o_ref[...] = 1.0 / (1.0 + jnp.exp(-t))

    return pl.pallas_call(
        body,
        grid=(nsteps,),
        in_specs=[pl.BlockSpec((2, _BT, H), lambda i: (0, i, 0)),
                  pl.BlockSpec((_BT, H), lambda i: (i, 0)),
                  pl.BlockSpec((_BT, 1), lambda i: (i, 0)),
                  pl.BlockSpec((1, H), lambda i: (0, 0)),
                  pl.BlockSpec((_BT, H), lambda i: (i, 0)),
                  pl.BlockSpec((1, 1), lambda i: (0, 0))],
        out_specs=pl.BlockSpec((1, 1), lambda i: (0, 0)),
        out_shape=jax.ShapeDtypeStruct((1, 1), jnp.float32),
    )(acc, yi, dis2, bd, Wfc2d, bfc)


def kernel(features, edge_index, edge_attr, W1, b1, W2, b2, Wd, bd, Wfc, bfc):
    pad = EP - E
    src = jnp.concatenate(
        [edge_index[0].astype(jnp.int32), jnp.zeros((pad,), jnp.int32)]
    ).reshape(NW, NCH, CH)
    dst = jnp.concatenate(
        [edge_index[1].astype(jnp.int32), jnp.zeros((pad,), jnp.int32)]
    ).reshape(NW, NCH, CH)
    w = jnp.concatenate(
        [edge_attr.astype(jnp.float32), jnp.zeros((pad,), jnp.float32)]
    ).reshape(NW, NCH, CH)
    xP = jnp.pad(features, ((0, NP - N), (0, 0)))
    WfcP = jnp.pad(Wfc.reshape(N, H), ((0, NP - N), (0, 0)))
    b1r = b1.reshape(1, H)
    b2r = b2.reshape(1, F)
    bdr = bd.reshape(1, H)
    bfcr = bfc.reshape(1, 1)

    degp = _sc_deg(dst, w)
    h1 = _mm(xP, W1)
    h1p, dis, invd = _norms(degp, h1)
    acc1 = _sc_conv64(h1p, src, dst, w)
    h2p, h2i = _post1(acc1, h1, dis, invd, b1r, W2)
    acc2 = _sc_conv128(h2p, src, dst, w)
    z, y = _post2(acc2, h2i, dis, b2r, xP, Wd)
    probs, degp2 = _sc_decode(z, src, dst)
    yp, yi, dis2 = _norms2(degp2, y)
    acc3 = _sc_conv64(yp, src, dst, probs)
    out = _head(acc3, yi, dis2, bdr, WfcP, bfcr)
    return out[0, 0]
